# Initial kernel scaffold; baseline (speedup 1.0000x reference)
import jax
import jax.numpy as jnp
from jax.experimental import pallas as pl


def kernel(x, edge_index, W0, b0, W1, b1, W2, b2):
    raise NotImplementedError("write your pallas kernel here")


# speedup vs baseline: 3.1614x; 3.1614x over previous
import functools

import jax
import jax.numpy as jnp
from jax import lax
from jax.experimental import pallas as pl
from jax.experimental.pallas import tpu as pltpu
from jax.experimental.pallas import tpu_sc as plsc

N = 10000
E = 320000
D = 128

NC = 2
NS = 16
EPT = E // NS
CH = 80
GC = EPT // CH
RPT = 624
TAIL = N - RPT * NS

assert EPT * NS == E and GC * CH == EPT and RPT % 8 == 0 and TAIL % 8 == 0


def _msg_body(h_hbm, src_hbm, dst_hbm, zrows_hbm, accf_hbm, accb_hbm,
              gidx_v, sidx_v, rows_v, acc_sh, sem):
    c = lax.axis_index("c")
    s = lax.axis_index("s")
    rows = pl.ds(pl.multiple_of(s * RPT, 8), RPT)
    tail = pl.ds(RPT * NS, TAIL)

    pltpu.sync_copy(zrows_hbm.at[pl.ds(0, RPT)], acc_sh.at[rows])

    @pl.when(s == NS - 1)
    def _ztail():
        pltpu.sync_copy(zrows_hbm.at[pl.ds(0, TAIL)], acc_sh.at[tail])

    plsc.subcore_barrier()

    def run_dir(gat_hbm, sct_hbm):
        def chunk(g, carry):
            base = pl.multiple_of(s * EPT + g * CH, 8)
            pltpu.sync_copy(gat_hbm.at[pl.ds(base, CH)], gidx_v)
            pltpu.sync_copy(sct_hbm.at[pl.ds(base, CH)], sidx_v)
            pltpu.async_copy(h_hbm.at[gidx_v], rows_v, sem).wait()
            pltpu.sync_copy(rows_v, acc_sh.at[sidx_v], add=True)
            return carry
        lax.fori_loop(0, GC, chunk, 0)

    @pl.when(c == 0)
    def _fwd():
        run_dir(src_hbm, dst_hbm)

    @pl.when(c == 1)
    def _bwd():
        run_dir(dst_hbm, src_hbm)

    plsc.subcore_barrier()

    @pl.when(c == 0)
    def _outf():
        pltpu.sync_copy(acc_sh.at[rows], accf_hbm.at[rows])

        @pl.when(s == NS - 1)
        def _outf_tail():
            pltpu.sync_copy(acc_sh.at[tail], accf_hbm.at[tail])

    @pl.when(c == 1)
    def _outb():
        pltpu.sync_copy(acc_sh.at[rows], accb_hbm.at[rows])

        @pl.when(s == NS - 1)
        def _outb_tail():
            pltpu.sync_copy(acc_sh.at[tail], accb_hbm.at[tail])


_msg = pl.kernel(
    _msg_body,
    mesh=plsc.VectorSubcoreMesh(core_axis_name="c", subcore_axis_name="s"),
    out_type=(
        jax.ShapeDtypeStruct((N, D), jnp.float32),
        jax.ShapeDtypeStruct((N, D), jnp.float32),
    ),
    scratch_types=[
        pltpu.VMEM((CH,), jnp.int32),
        pltpu.VMEM((CH,), jnp.int32),
        pltpu.VMEM((CH, D), jnp.float32),
        pltpu.VMEM_SHARED((N, D), jnp.float32),
        pltpu.SemaphoreType.DMA,
    ],
)


def _combine_body(h_ref, af_ref, ab_ref, df_ref, db_ref, o_ref):
    invf = 0.5 / jnp.maximum(df_ref[...], 1.0)
    invb = 0.5 / jnp.maximum(db_ref[...], 1.0)
    o_ref[...] = h_ref[...] + af_ref[...] * invf + ab_ref[...] * invb


def _combine_dense_body(h_ref, af_ref, ab_ref, df_ref, db_ref, w_ref, b_ref,
                        o_ref):
    invf = 0.5 / jnp.maximum(df_ref[...], 1.0)
    invb = 0.5 / jnp.maximum(db_ref[...], 1.0)
    c = h_ref[...] + af_ref[...] * invf + ab_ref[...] * invb
    y = jnp.dot(c, w_ref[...], preferred_element_type=jnp.float32,
                precision=lax.Precision.HIGHEST) + b_ref[...]
    o_ref[...] = c + jnp.maximum(y, 0.0)


_R = 2000

_row_spec = pl.BlockSpec((_R, D), lambda i: (i, 0))
_deg_spec = pl.BlockSpec((_R, 1), lambda i: (i, 0))

_combine = pl.pallas_call(
    _combine_body,
    grid=(N // _R,),
    in_specs=[_row_spec, _row_spec, _row_spec, _deg_spec, _deg_spec],
    out_specs=_row_spec,
    out_shape=jax.ShapeDtypeStruct((N, D), jnp.float32),
)

_combine_dense = pl.pallas_call(
    _combine_dense_body,
    grid=(N // _R,),
    in_specs=[_row_spec, _row_spec, _row_spec, _deg_spec, _deg_spec,
              pl.BlockSpec((D, D), lambda i: (0, 0)),
              pl.BlockSpec((1, D), lambda i: (0, 0))],
    out_specs=_row_spec,
    out_shape=jax.ShapeDtypeStruct((N, D), jnp.float32),
)


def kernel(x, edge_index, W0, b0, W1, b1, W2, b2):
    src = edge_index[0]
    dst = edge_index[1]
    zrows = jnp.zeros((RPT, D), jnp.float32)
    ones = jnp.ones((N, D), jnp.float32)

    degf_full, degb_full = _msg(ones, src, dst, zrows)
    degf = degf_full[:, :1]
    degb = degb_full[:, :1]

    h = x
    for W, b in ((W0, b0), (W1, b1), (W2, b2)):
        af, ab = _msg(h, src, dst, zrows)
        h = _combine(h, af, ab, degf, degb)
        af, ab = _msg(h, src, dst, zrows)
        h = _combine_dense(h, af, ab, degf, degb, W, b.reshape(1, D))
    return h


# speedup vs baseline: 6.2913x; 1.9900x over previous
import jax
import jax.numpy as jnp
from jax import lax
from jax.experimental import pallas as pl
from jax.experimental.pallas import tpu as pltpu
from jax.experimental.pallas import tpu_sc as plsc

N = 10000
E = 320000
D = 128

NC = 2
NS = 16
EPT = E // NS
CH = 80
GC = EPT // CH
RPT = 624
TAIL = N - RPT * NS

assert EPT * NS == E and GC * CH == EPT and RPT % 8 == 0 and TAIL % 8 == 0

_mesh = plsc.VectorSubcoreMesh(core_axis_name="c", subcore_axis_name="s")


def _msg_body(h_hbm, ei_hbm, zrows_hbm, accf_hbm, accb_hbm,
              gidx_v, sidx_v, rows_v, acc_sh, gsem, isem):
    c = lax.axis_index("c")
    s = lax.axis_index("s")
    rows = pl.ds(pl.multiple_of(s * RPT, 8), RPT)
    tail = pl.ds(RPT * NS, TAIL)

    pltpu.sync_copy(zrows_hbm.at[pl.ds(0, RPT)], acc_sh.at[rows])

    @pl.when(s == NS - 1)
    def _ztail():
        pltpu.sync_copy(zrows_hbm.at[pl.ds(0, TAIL)], acc_sh.at[tail])

    def eslice(d, g):
        return pl.ds(pl.multiple_of(d * E + s * EPT + g * CH, 8), CH)

    def idx_copies(g, b):
        return (pltpu.make_async_copy(ei_hbm.at[eslice(c, g)],
                                      gidx_v.at[b], isem),
                pltpu.make_async_copy(ei_hbm.at[eslice(1 - c, g)],
                                      sidx_v.at[b], isem))

    plsc.subcore_barrier()

    for cp in idx_copies(0, 0):
        cp.start()
        cp.wait()
    pltpu.async_copy(h_hbm.at[gidx_v.at[0]], rows_v.at[0], gsem)
    for cp in idx_copies(1, 1):
        cp.start()

    def chunk(g, carry):
        b = lax.rem(g, 2)
        pltpu.make_async_copy(h_hbm.at[gidx_v.at[b]], rows_v.at[b], gsem).wait()

        @pl.when(g + 1 < GC)
        def _next():
            for cp in idx_copies(g + 1, 1 - b):
                cp.wait()
            pltpu.async_copy(h_hbm.at[gidx_v.at[1 - b]], rows_v.at[1 - b], gsem)

        pltpu.sync_copy(rows_v.at[b], acc_sh.at[sidx_v.at[b]], add=True)

        @pl.when(g + 2 < GC)
        def _pref():
            for cp in idx_copies(g + 2, b):
                cp.start()

        return carry

    lax.fori_loop(0, GC, chunk, 0)
    plsc.subcore_barrier()

    @pl.when(c == 0)
    def _outf():
        pltpu.sync_copy(acc_sh.at[rows], accf_hbm.at[rows])

        @pl.when(s == NS - 1)
        def _outf_tail():
            pltpu.sync_copy(acc_sh.at[tail], accf_hbm.at[tail])

    @pl.when(c == 1)
    def _outb():
        pltpu.sync_copy(acc_sh.at[rows], accb_hbm.at[rows])

        @pl.when(s == NS - 1)
        def _outb_tail():
            pltpu.sync_copy(acc_sh.at[tail], accb_hbm.at[tail])


_msg = pl.kernel(
    _msg_body,
    mesh=_mesh,
    out_type=(
        jax.ShapeDtypeStruct((N, D), jnp.float32),
        jax.ShapeDtypeStruct((N, D), jnp.float32),
    ),
    scratch_types=[
        pltpu.VMEM((2, CH), jnp.int32),
        pltpu.VMEM((2, CH), jnp.int32),
        pltpu.VMEM((2, CH, D), jnp.float32),
        pltpu.VMEM_SHARED((N, D), jnp.float32),
        pltpu.SemaphoreType.DMA,
        pltpu.SemaphoreType.DMA,
    ],
)


def _combine_body(h_ref, af_ref, ab_ref, df_ref, db_ref, o_ref):
    invf = 0.5 / jnp.maximum(df_ref[...], 1.0)
    invb = 0.5 / jnp.maximum(db_ref[...], 1.0)
    o_ref[...] = h_ref[...] + af_ref[...] * invf + ab_ref[...] * invb


def _combine_dense_body(h_ref, af_ref, ab_ref, df_ref, db_ref, w_ref, b_ref,
                        o_ref):
    invf = 0.5 / jnp.maximum(df_ref[...], 1.0)
    invb = 0.5 / jnp.maximum(db_ref[...], 1.0)
    c = h_ref[...] + af_ref[...] * invf + ab_ref[...] * invb
    y = jnp.dot(c, w_ref[...], preferred_element_type=jnp.float32,
                precision=lax.Precision.HIGHEST) + b_ref[...]
    o_ref[...] = c + jnp.maximum(y, 0.0)


_R = 2000

_row_spec = pl.BlockSpec((_R, D), lambda i: (i, 0))
_deg_spec = pl.BlockSpec((_R, 1), lambda i: (i, 0))

_combine = pl.pallas_call(
    _combine_body,
    grid=(N // _R,),
    in_specs=[_row_spec, _row_spec, _row_spec, _deg_spec, _deg_spec],
    out_specs=_row_spec,
    out_shape=jax.ShapeDtypeStruct((N, D), jnp.float32),
)

_combine_dense = pl.pallas_call(
    _combine_dense_body,
    grid=(N // _R,),
    in_specs=[_row_spec, _row_spec, _row_spec, _deg_spec, _deg_spec,
              pl.BlockSpec((D, D), lambda i: (0, 0)),
              pl.BlockSpec((1, D), lambda i: (0, 0))],
    out_specs=_row_spec,
    out_shape=jax.ShapeDtypeStruct((N, D), jnp.float32),
)


def kernel(x, edge_index, W0, b0, W1, b1, W2, b2):
    ei_flat = edge_index.reshape(2 * E)
    zrows = jnp.zeros((RPT, D), jnp.float32)
    ones = jnp.ones((N, D), jnp.float32)

    degf_full, degb_full = _msg(ones, ei_flat, zrows)
    degf = degf_full[:, :1]
    degb = degb_full[:, :1]

    h = x
    for W, b in ((W0, b0), (W1, b1), (W2, b2)):
        af, ab = _msg(h, ei_flat, zrows)
        h = _combine(h, af, ab, degf, degb)
        af, ab = _msg(h, ei_flat, zrows)
        h = _combine_dense(h, af, ab, degf, degb, W, b.reshape(1, D))
    return h
